# Initial kernel scaffold; baseline (speedup 1.0000x reference)
import jax
import jax.numpy as jnp
from jax.experimental import pallas as pl


def kernel(x, edge_index, batch, params):
    raise NotImplementedError("write your pallas kernel here")


# speedup vs baseline: 11.2939x; 11.2939x over previous
import functools

import jax
import jax.numpy as jnp
from jax import lax
from jax.experimental import pallas as pl
from jax.experimental.pallas import tpu as pltpu
from jax.experimental.pallas import tpu_sc as plsc

N = 10000
F = 128
DIM = 32
G = 64
C = 10
E = 320000
L = 3

NPAD = 10112
TRASH = N
NC = 2
NS = 16
MACROS = 10
GCH = 8
GPT = MACROS * GCH
E_PAD = NC * NS * GPT * 128
EG = E_PAD // 128
RPT = NPAD // NS

_mesh = plsc.VectorSubcoreMesh(core_axis_name="c", subcore_axis_name="s")


@functools.partial(
    pl.kernel,
    out_type=jax.ShapeDtypeStruct((NC, NPAD, DIM), jnp.float32),
    mesh=_mesh,
    scratch_types=[
        pltpu.VMEM((GCH, 128), jnp.int32),
        pltpu.VMEM((GCH, 128), jnp.int32),
        pltpu.VMEM((GCH, 128, DIM), jnp.float32),
        pltpu.VMEM_SHARED((NPAD, DIM), jnp.float32),
        pltpu.SemaphoreType.DMA,
    ],
    compiler_params=pltpu.CompilerParams(use_tc_tiling_on_sc=False),
)
def _sc_agg(row_hbm, col_hbm, hw_hbm, out_hbm, colbuf, rowbuf, vals, acc, sem):
    cid = lax.axis_index("c")
    sid = lax.axis_index("s")
    wid = sid * NC + cid
    pltpu.sync_copy(hw_hbm.at[pl.ds(sid * RPT, RPT)],
                    acc.at[pl.ds(sid * RPT, RPT)])
    plsc.subcore_barrier()

    def macro(m, carry):
        gbase = wid * GPT + m * GCH
        pltpu.sync_copy(col_hbm.at[pl.ds(gbase, GCH)], colbuf)
        pltpu.sync_copy(row_hbm.at[pl.ds(gbase, GCH)], rowbuf)
        for g in range(GCH):
            for j in range(128 // 16):
                r = rowbuf[g, pl.ds(16 * j, 16)]
                c = colbuf[g, pl.ds(16 * j, 16)]
                rowbuf[g, pl.ds(16 * j, 16)] = jnp.where(r == c, TRASH, r)
        for g in range(GCH):
            pltpu.async_copy(hw_hbm.at[colbuf.at[g]], vals.at[g], sem).wait()
            pltpu.sync_copy(vals.at[g], acc.at[rowbuf.at[g]], add=True)
        return carry

    lax.fori_loop(0, MACROS, macro, 0)
    plsc.subcore_barrier()
    pltpu.sync_copy(acc.at[pl.ds(sid * RPT, RPT)],
                    out_hbm.at[cid, pl.ds(sid * RPT, RPT)])


def _segment_mean(batch2d, h):
    seg = lax.broadcasted_iota(jnp.int32, (1, G), 1)
    P = (batch2d == seg).astype(jnp.float32)
    s = lax.dot_general(P, h, (((0,), (0,)), ((), ())),
                        preferred_element_type=jnp.float32)
    cnt = jnp.sum(P, axis=0)[:, None]
    return s / jnp.maximum(cnt, 1.0)


def _head_body(x_ref, w_ref, batch_ref, hw_ref, pool_ref):
    x = x_ref[...]
    hw_ref[:N, :] = jnp.dot(x, w_ref[...], preferred_element_type=jnp.float32)
    hw_ref[N:, :] = jnp.zeros((NPAD - N, DIM), jnp.float32)
    pool_ref[...] = _segment_mean(batch_ref[...], x)


_head = pl.pallas_call(
    _head_body,
    out_shape=(jax.ShapeDtypeStruct((NPAD, DIM), jnp.float32),
               jax.ShapeDtypeStruct((G, F), jnp.float32)),
)


def _bn_relu(t, gamma, beta):
    u = jnp.maximum(t, 0.0)
    mu = jnp.mean(u, axis=0, keepdims=True)
    var = jnp.mean(u * u, axis=0, keepdims=True) - mu * mu
    return gamma * (u - mu) / jnp.sqrt(var + 1e-5) + beta


def _mlp_body(p0_ref, p1_ref, hw_ref, b0_ref, g0_ref, be0_ref,
              w1_ref, b1_ref, g1_ref, be1_ref, wn_ref, batch_ref,
              hwn_ref, pool_ref):
    agg = p0_ref[:N, :] + p1_ref[:N, :] - hw_ref[:N, :]
    h = _bn_relu(agg + b0_ref[...], g0_ref[...], be0_ref[...])
    t = jnp.dot(h, w1_ref[...], preferred_element_type=jnp.float32) + b1_ref[...]
    h2 = _bn_relu(t, g1_ref[...], be1_ref[...])
    hwn_ref[:N, :] = jnp.dot(h2, wn_ref[...], preferred_element_type=jnp.float32)
    hwn_ref[N:, :] = jnp.zeros((NPAD - N, DIM), jnp.float32)
    pool_ref[...] = _segment_mean(batch_ref[...], h2)


_mlp = pl.pallas_call(
    _mlp_body,
    out_shape=(jax.ShapeDtypeStruct((NPAD, DIM), jnp.float32),
               jax.ShapeDtypeStruct((G, DIM), jnp.float32)),
)


def _cls_body(px_ref, q0_ref, q1_ref, q2_ref, wc1_ref, bc1_ref,
              wc2_ref, bc2_ref, out_ref):
    z = jnp.concatenate(
        [px_ref[...], q0_ref[...], q1_ref[...], q2_ref[...]], axis=1)
    h = jnp.maximum(
        jnp.dot(z, wc1_ref[...], preferred_element_type=jnp.float32)
        + bc1_ref[...], 0.0)
    o = (jnp.dot(h, wc2_ref[...], preferred_element_type=jnp.float32)
         + bc2_ref[...])
    m = jnp.max(o, axis=1, keepdims=True)
    e = jnp.exp(o - m)
    out_ref[...] = (o - m) - jnp.log(jnp.sum(e, axis=1, keepdims=True))


_cls = pl.pallas_call(
    _cls_body,
    out_shape=jax.ShapeDtypeStruct((G, C), jnp.float32),
)


def kernel(x, edge_index, batch, params):
    row = edge_index[0]
    col = edge_index[1]
    pad = E_PAD - E
    row2d = jnp.concatenate(
        [row, jnp.full((pad,), TRASH, jnp.int32)]).reshape(EG, 128)
    col2d = jnp.concatenate(
        [col, jnp.zeros((pad,), jnp.int32)]).reshape(EG, 128)
    batch2d = batch[:, None]
    p = params

    hw, poolx = _head(x, p["W0_0"], batch2d)
    pools = [poolx]
    for k in range(L):
        parts = _sc_agg(row2d, col2d, hw)
        wn = p[f"W{k + 1}_0"] if k < L - 1 else p["W1_0"]
        hw, pk = _mlp(
            parts[0], parts[1], hw,
            p[f"b{k}_0"][None], p[f"g{k}_0"][None], p[f"be{k}_0"][None],
            p[f"W{k}_1"],
            p[f"b{k}_1"][None], p[f"g{k}_1"][None], p[f"be{k}_1"][None],
            wn, batch2d)
        pools.append(pk)
    return _cls(pools[0], pools[1], pools[2], pools[3],
                p["Wc1"], p["bc1"][None], p["Wc2"], p["bc2"][None])


# speedup vs baseline: 14.9675x; 1.3253x over previous
import functools

import jax
import jax.numpy as jnp
from jax import lax
from jax.experimental import pallas as pl
from jax.experimental.pallas import tpu as pltpu
from jax.experimental.pallas import tpu_sc as plsc

N = 10000
F = 128
DIM = 32
G = 64
C = 10
E = 320000
L = 3

NPAD = 10112
TRASH = N
NC = 2
NS = 16
MACROS = 10
GCH = 8
GPT = MACROS * GCH
E_PAD = NC * NS * GPT * 128
EG = E_PAD // 128
RPT = NPAD // NS

_mesh = plsc.VectorSubcoreMesh(core_axis_name="c", subcore_axis_name="s")


@functools.partial(
    pl.kernel,
    out_type=jax.ShapeDtypeStruct((NC, NPAD, DIM), jnp.float32),
    mesh=_mesh,
    scratch_types=[
        pltpu.VMEM((2, GCH, 128), jnp.int32),
        pltpu.VMEM((2, GCH, 128), jnp.int32),
        pltpu.VMEM((2, GCH, 128, DIM), jnp.float32),
        pltpu.VMEM_SHARED((NPAD, DIM), jnp.float32),
        pltpu.SemaphoreType.DMA,
        pltpu.SemaphoreType.DMA,
    ],
    compiler_params=pltpu.CompilerParams(use_tc_tiling_on_sc=False),
)
def _sc_agg(row_hbm, col_hbm, hw_hbm, out_hbm, colbuf, rowbuf, vals, acc,
            sem_g, sem_s):
    cid = lax.axis_index("c")
    sid = lax.axis_index("s")
    wid = sid * NC + cid
    pltpu.sync_copy(hw_hbm.at[pl.ds(sid * RPT, RPT)],
                    acc.at[pl.ds(sid * RPT, RPT)])
    plsc.subcore_barrier()

    def fire(m, b):
        gbase = wid * GPT + m * GCH
        pltpu.sync_copy(col_hbm.at[pl.ds(gbase, GCH)], colbuf.at[b])
        pltpu.sync_copy(row_hbm.at[pl.ds(gbase, GCH)], rowbuf.at[b])
        for g in range(GCH):
            pltpu.async_copy(hw_hbm.at[colbuf.at[b, g]], vals.at[b, g], sem_g)
        for g in range(GCH):
            for j in range(128 // 16):
                r = rowbuf[b, g, pl.ds(16 * j, 16)]
                c = colbuf[b, g, pl.ds(16 * j, 16)]
                rowbuf[b, g, pl.ds(16 * j, 16)] = jnp.where(r == c, TRASH, r)

    def finish(b):
        for g in range(GCH):
            pltpu.make_async_copy(
                hw_hbm.at[colbuf.at[b, g]], vals.at[b, g], sem_g).wait()
        for g in range(GCH):
            pltpu.async_copy(vals.at[b, g], acc.at[rowbuf.at[b, g]], sem_s,
                             add=True)

    def drain_scatters(b):
        for g in range(GCH):
            pltpu.make_async_copy(
                vals.at[b, g], acc.at[rowbuf.at[b, g]], sem_s).wait()

    for m in range(MACROS):
        b = m % 2
        if m >= 2:
            drain_scatters(b)
        fire(m, b)
        if m >= 1:
            finish(1 - b)
    finish((MACROS - 1) % 2)
    drain_scatters(0)
    drain_scatters(1)
    plsc.subcore_barrier()
    pltpu.sync_copy(acc.at[pl.ds(sid * RPT, RPT)],
                    out_hbm.at[cid, pl.ds(sid * RPT, RPT)])


def _segment_mean(batch2d, h):
    seg = lax.broadcasted_iota(jnp.int32, (1, G), 1)
    P = (batch2d == seg).astype(jnp.float32)
    s = lax.dot_general(P, h, (((0,), (0,)), ((), ())),
                        preferred_element_type=jnp.float32)
    cnt = jnp.sum(P, axis=0)[:, None]
    return s / jnp.maximum(cnt, 1.0)


def _head_body(x_ref, w_ref, batch_ref, hw_ref, pool_ref):
    x = x_ref[...]
    hw_ref[:N, :] = jnp.dot(x, w_ref[...], preferred_element_type=jnp.float32)
    hw_ref[N:, :] = jnp.zeros((NPAD - N, DIM), jnp.float32)
    pool_ref[...] = _segment_mean(batch_ref[...], x)


_head = pl.pallas_call(
    _head_body,
    out_shape=(jax.ShapeDtypeStruct((NPAD, DIM), jnp.float32),
               jax.ShapeDtypeStruct((G, F), jnp.float32)),
)


def _bn_relu(t, gamma, beta):
    u = jnp.maximum(t, 0.0)
    mu = jnp.mean(u, axis=0, keepdims=True)
    var = jnp.mean(u * u, axis=0, keepdims=True) - mu * mu
    return gamma * (u - mu) / jnp.sqrt(var + 1e-5) + beta


def _mlp_body(p0_ref, p1_ref, hw_ref, b0_ref, g0_ref, be0_ref,
              w1_ref, b1_ref, g1_ref, be1_ref, wn_ref, batch_ref,
              hwn_ref, pool_ref):
    agg = p0_ref[:N, :] + p1_ref[:N, :] - hw_ref[:N, :]
    h = _bn_relu(agg + b0_ref[...], g0_ref[...], be0_ref[...])
    t = jnp.dot(h, w1_ref[...], preferred_element_type=jnp.float32) + b1_ref[...]
    h2 = _bn_relu(t, g1_ref[...], be1_ref[...])
    hwn_ref[:N, :] = jnp.dot(h2, wn_ref[...], preferred_element_type=jnp.float32)
    hwn_ref[N:, :] = jnp.zeros((NPAD - N, DIM), jnp.float32)
    pool_ref[...] = _segment_mean(batch_ref[...], h2)


_mlp = pl.pallas_call(
    _mlp_body,
    out_shape=(jax.ShapeDtypeStruct((NPAD, DIM), jnp.float32),
               jax.ShapeDtypeStruct((G, DIM), jnp.float32)),
)


def _cls_body(px_ref, q0_ref, q1_ref, q2_ref, wc1_ref, bc1_ref,
              wc2_ref, bc2_ref, out_ref):
    z = jnp.concatenate(
        [px_ref[...], q0_ref[...], q1_ref[...], q2_ref[...]], axis=1)
    h = jnp.maximum(
        jnp.dot(z, wc1_ref[...], preferred_element_type=jnp.float32)
        + bc1_ref[...], 0.0)
    o = (jnp.dot(h, wc2_ref[...], preferred_element_type=jnp.float32)
         + bc2_ref[...])
    m = jnp.max(o, axis=1, keepdims=True)
    e = jnp.exp(o - m)
    out_ref[...] = (o - m) - jnp.log(jnp.sum(e, axis=1, keepdims=True))


_cls = pl.pallas_call(
    _cls_body,
    out_shape=jax.ShapeDtypeStruct((G, C), jnp.float32),
)


def kernel(x, edge_index, batch, params):
    row = edge_index[0]
    col = edge_index[1]
    pad = E_PAD - E
    row2d = jnp.concatenate(
        [row, jnp.full((pad,), TRASH, jnp.int32)]).reshape(EG, 128)
    col2d = jnp.concatenate(
        [col, jnp.zeros((pad,), jnp.int32)]).reshape(EG, 128)
    batch2d = batch[:, None]
    p = params

    hw, poolx = _head(x, p["W0_0"], batch2d)
    pools = [poolx]
    for k in range(L):
        parts = _sc_agg(row2d, col2d, hw)
        wn = p[f"W{k + 1}_0"] if k < L - 1 else p["W1_0"]
        hw, pk = _mlp(
            parts[0], parts[1], hw,
            p[f"b{k}_0"][None], p[f"g{k}_0"][None], p[f"be{k}_0"][None],
            p[f"W{k}_1"],
            p[f"b{k}_1"][None], p[f"g{k}_1"][None], p[f"be{k}_1"][None],
            wn, batch2d)
        pools.append(pk)
    return _cls(pools[0], pools[1], pools[2], pools[3],
                p["Wc1"], p["bc1"][None], p["Wc2"], p["bc2"][None])
